# baseline (speedup 1.0000x reference)
import functools

import jax
import jax.numpy as jnp
import numpy as np
from jax import lax
from jax.experimental import pallas as pl
from jax.experimental.pallas import tpu as pltpu
from jax.experimental.pallas import tpu_sc as plsc

_SIZE = 32000
_N = 4096
_PADDING_IDX = 0
_SMOOTHING = 0.1
_CONFIDENCE = 1.0 - _SMOOTHING
_EPS = np.float32(_SMOOTHING / (_SIZE - 2))
_C1 = np.float32((_SIZE - 2) * float(_EPS) * np.log(float(_EPS)))
_CLOGC = np.float32(_CONFIDENCE * np.log(_CONFIDENCE))
_CONF_F = np.float32(_CONFIDENCE)

_NSC = 1024
_NTC = _N - _NSC
_LANES = 16
_GATHER_W = 128


def _lane_take(v, perm):
    dnums = lax.GatherDimensionNumbers(
        offset_dims=(), collapsed_slice_dims=(0,), start_index_map=(0,))
    return lax.gather(v, perm[:, None], dnums, (1,),
                      mode=lax.GatherScatterMode.PROMISE_IN_BOUNDS)


def _sc_body(gt_hbm, x16_hbm, idx_hbm, lane_hbm, rows_hbm,
             gt_buf, idx_buf, blk_buf, lane_buf, rows_buf, dma_sem,
             *, rpw, n_steps):
    wid = lax.axis_index("s") * 2 + lax.axis_index("c")
    base = _NTC + wid * rpw
    iota = lax.iota(jnp.int32, _LANES)
    neg_inf = jnp.full((_LANES,), -jnp.inf, dtype=jnp.float32)
    zeros_i = jnp.zeros((_LANES,), dtype=jnp.int32)
    big = jnp.int32(2147483647)

    n_grp = rpw // _LANES
    for g in range(n_grp):
        def row_fn(rr, acc):
            pltpu.sync_copy(gt_hbm.at[base + g * _LANES + rr], gt_buf)

            def step(j, mc):
                m, vidx = mc
                v = gt_buf[pl.ds(j * _LANES, _LANES)]
                cur = iota + j * _LANES
                better = v > m
                return (jnp.where(better, v, m), jnp.where(better, cur, vidx))

            m, vidx = lax.fori_loop(0, n_steps, step, (neg_inf, zeros_i))
            for k in (1, 2, 4, 8):
                perm = jnp.bitwise_xor(iota, k)
                m2 = _lane_take(m, perm)
                v2 = _lane_take(vidx, perm)
                take = (m2 > m) | ((m2 == m) & (v2 < vidx))
                m = jnp.where(take, m2, m)
                vidx = jnp.where(take, v2, vidx)
            return jnp.where(iota == rr, vidx, acc)

        tvec = lax.fori_loop(0, _LANES, row_fn, zeros_i)
        idx_buf[pl.ds(g * _LANES, _LANES)] = tvec
        rowvec = iota + (base + g * _LANES)
        flat = rowvec * _SIZE + tvec
        blk_buf[pl.ds(g * _LANES, _LANES)] = lax.shift_right_logical(flat, 7)
        lane_buf[pl.ds(g * _LANES, _LANES)] = lax.bitwise_and(flat, 127)

    pltpu.async_copy(x16_hbm.at[blk_buf], rows_buf, dma_sem).wait()

    pltpu.sync_copy(idx_buf, idx_hbm.at[pl.ds(base, rpw)])
    pltpu.sync_copy(lane_buf, lane_hbm.at[pl.ds(base, rpw)])
    pltpu.sync_copy(rows_buf, rows_hbm.at[pl.ds(base, rpw)])


def _sc_argmax(gt, x16):
    rpw = _NSC // 32
    body = functools.partial(_sc_body, rpw=rpw, n_steps=_SIZE // _LANES)
    return pl.kernel(
        body,
        out_type=[
            jax.ShapeDtypeStruct((_N,), jnp.int32),
            jax.ShapeDtypeStruct((_N,), jnp.int32),
            jax.ShapeDtypeStruct((_N, _GATHER_W), jnp.float32),
        ],
        mesh=plsc.VectorSubcoreMesh(core_axis_name="c", subcore_axis_name="s"),
        scratch_types=[
            pltpu.VMEM((_SIZE,), jnp.float32),
            pltpu.VMEM((rpw,), jnp.int32),
            pltpu.VMEM((rpw,), jnp.int32),
            pltpu.VMEM((rpw,), jnp.int32),
            pltpu.VMEM((rpw, _GATHER_W), jnp.float32),
            pltpu.SemaphoreType.DMA,
        ],
    )(gt, x16)


def _tc_body(x_ref, gt_ref, s_ref, x0_ref, idx_ref, xat_ref, m_ref,
             *, n_col_blocks, row_block, tc_row_blocks):
    j = pl.program_id(0)
    i = pl.program_id(1)
    rows = pl.ds(i * row_block, row_block)

    x_tile = x_ref[...]
    ts = jnp.sum(x_tile, axis=1, keepdims=True)

    @pl.when(j == 0)
    def _():
        s_ref[rows, :] = ts
        x0_ref[rows, :] = x_tile[:, 0:1]

    @pl.when(j != 0)
    def _():
        s_ref[rows, :] = s_ref[rows, :] + ts

    @pl.when(i < tc_row_blocks)
    def _gt():
        gt_tile = gt_ref[...]
        cols = gt_tile.shape[1]
        tm = jnp.max(gt_tile, axis=1, keepdims=True)
        ta = jnp.argmax(gt_tile, axis=1)[:, None] + j * cols
        onehot = jax.lax.broadcasted_iota(jnp.int32, gt_tile.shape, 1) == (
            ta - j * cols)
        txat = jnp.sum(jnp.where(onehot, x_tile, 0.0), axis=1, keepdims=True)

        @pl.when(j == 0)
        def _():
            m_ref[rows, :] = tm
            idx_ref[rows, :] = ta
            xat_ref[rows, :] = txat

        @pl.when(j != 0)
        def _():
            better = tm > m_ref[rows, :]
            m_ref[rows, :] = jnp.where(better, tm, m_ref[rows, :])
            idx_ref[rows, :] = jnp.where(better, ta, idx_ref[rows, :])
            xat_ref[rows, :] = jnp.where(better, txat, xat_ref[rows, :])


def _tc_sums(x, gt):
    row_block = 256
    col_block = 3200
    n_row_blocks = _N // row_block
    n_col_blocks = _SIZE // col_block
    tc_row_blocks = _NTC // row_block

    body = functools.partial(
        _tc_body,
        n_col_blocks=n_col_blocks,
        row_block=row_block,
        tc_row_blocks=tc_row_blocks,
    )
    full = pl.BlockSpec((_N, 1), lambda j, i: (0, 0))
    return pl.pallas_call(
        body,
        grid=(n_col_blocks, n_row_blocks),
        in_specs=[
            pl.BlockSpec((row_block, col_block), lambda j, i: (i, j)),
            pl.BlockSpec((row_block, col_block),
                         lambda j, i: (jnp.minimum(i, tc_row_blocks - 1), j)),
        ],
        out_specs=[full, full, full, full],
        out_shape=[
            jax.ShapeDtypeStruct((_N, 1), jnp.float32),
            jax.ShapeDtypeStruct((_N, 1), jnp.float32),
            jax.ShapeDtypeStruct((_N, 1), jnp.int32),
            jax.ShapeDtypeStruct((_N, 1), jnp.float32),
        ],
        scratch_shapes=[pltpu.VMEM((_N, 1), jnp.float32)],
    )(x, gt)


def _combine_body(s_ref, x0_ref, idxb_ref, xatb_ref, idxs_ref, lanes_ref,
                  rows_ref, out_ref):
    rowid = jax.lax.broadcasted_iota(jnp.int32, (_N, 1), 0)
    use_sc = rowid >= _NTC
    onehot = jax.lax.broadcasted_iota(jnp.int32, (_N, _GATHER_W), 1) == lanes_ref[...]
    xats = jnp.sum(jnp.where(onehot, rows_ref[...], 0.0), axis=1,
                   keepdims=True)
    idx = jnp.where(use_sc, idxs_ref[...], idxb_ref[...])
    xat = jnp.where(use_sc, xats, xatb_ref[...])
    contrib = (_C1 + _CLOGC) - _EPS * (s_ref[...] - x0_ref[...]) + (
        _EPS - _CONF_F) * xat
    contrib = jnp.where(idx == _PADDING_IDX, 0.0, contrib)
    inv_count = np.float32(1.0 / (_N * _SIZE))
    out_ref[...] = jnp.reshape(jnp.sum(contrib) * inv_count, (1, 1))


def _combine(s, x0, idxb, xatb, idxs, lanes, rows):
    return pl.pallas_call(
        _combine_body,
        out_shape=jax.ShapeDtypeStruct((1, 1), jnp.float32),
    )(s, x0, idxb, xatb, idxs, lanes, rows)


@jax.jit
def kernel(x, gt):
    x16 = jnp.reshape(x, (-1, _GATHER_W))
    idx_sc, lane_sc, rows_sc = _sc_argmax(gt, x16)
    s, x0, idx_b, xat_b = _tc_sums(x, gt)
    out = _combine(s, x0, idx_b, xat_b,
                   jnp.reshape(idx_sc, (_N, 1)),
                   jnp.reshape(lane_sc, (_N, 1)), rows_sc)
    return out[0, 0]


# speedup vs baseline: 1.0377x; 1.0377x over previous
import functools

import jax
import jax.numpy as jnp
import numpy as np
from jax import lax
from jax.experimental import pallas as pl
from jax.experimental.pallas import tpu as pltpu
from jax.experimental.pallas import tpu_sc as plsc

_SIZE = 32000
_N = 4096
_PADDING_IDX = 0
_SMOOTHING = 0.1
_CONFIDENCE = 1.0 - _SMOOTHING
_EPS = np.float32(_SMOOTHING / (_SIZE - 2))
_C1 = np.float32((_SIZE - 2) * float(_EPS) * np.log(float(_EPS)))
_CLOGC = np.float32(_CONFIDENCE * np.log(_CONFIDENCE))
_CONF_F = np.float32(_CONFIDENCE)

_NSC = 1024
_NTC = _N - _NSC
_LANES = 16
_GATHER_W = 128


def _lane_take(v, perm):
    dnums = lax.GatherDimensionNumbers(
        offset_dims=(), collapsed_slice_dims=(0,), start_index_map=(0,))
    return lax.gather(v, perm[:, None], dnums, (1,),
                      mode=lax.GatherScatterMode.PROMISE_IN_BOUNDS)


def _sc_body(gt_hbm, x16_hbm, idx_hbm, lane_hbm, rows_hbm,
             gt_buf0, gt_buf1, idx_buf, blk_buf, lane_buf, rows_buf,
             sem0, sem1, dma_sem, *, rpw, n_steps):
    wid = lax.axis_index("s") * 2 + lax.axis_index("c")
    base = _NTC + wid * rpw
    iota = lax.iota(jnp.int32, _LANES)
    neg_inf = jnp.full((_LANES,), -jnp.inf, dtype=jnp.float32)
    zeros_i = jnp.zeros((_LANES,), dtype=jnp.int32)

    bufs = (gt_buf0, gt_buf1)
    sems = (sem0, sem1)
    copies = [None, None]
    copies[0] = pltpu.async_copy(gt_hbm.at[base], bufs[0], sems[0])

    n_grp = rpw // _LANES
    for g in range(n_grp):
        acc = zeros_i
        for rr in range(_LANES):
            r = g * _LANES + rr
            nxt = (r + 1) % 2
            if r + 1 < rpw:
                copies[nxt] = pltpu.async_copy(
                    gt_hbm.at[base + r + 1], bufs[nxt], sems[nxt])
            copies[r % 2].wait()
            buf = bufs[r % 2]

            def step(j, mc, buf=buf):
                m, vidx = mc
                v = buf[pl.ds(j * _LANES, _LANES)]
                cur = iota + j * _LANES
                better = v > m
                return (jnp.where(better, v, m), jnp.where(better, cur, vidx))

            m, vidx = lax.fori_loop(0, n_steps, step, (neg_inf, zeros_i),
                                    unroll=8)
            for k in (1, 2, 4, 8):
                perm = jnp.bitwise_xor(iota, k)
                m2 = _lane_take(m, perm)
                v2 = _lane_take(vidx, perm)
                take = (m2 > m) | ((m2 == m) & (v2 < vidx))
                m = jnp.where(take, m2, m)
                vidx = jnp.where(take, v2, vidx)
            acc = jnp.where(iota == rr, vidx, acc)

        tvec = acc
        idx_buf[pl.ds(g * _LANES, _LANES)] = tvec
        rowvec = iota + (base + g * _LANES)
        flat = rowvec * _SIZE + tvec
        blk_buf[pl.ds(g * _LANES, _LANES)] = lax.shift_right_logical(flat, 7)
        lane_buf[pl.ds(g * _LANES, _LANES)] = lax.bitwise_and(flat, 127)

    pltpu.async_copy(x16_hbm.at[blk_buf], rows_buf, dma_sem).wait()

    pltpu.sync_copy(idx_buf, idx_hbm.at[pl.ds(base, rpw)])
    pltpu.sync_copy(lane_buf, lane_hbm.at[pl.ds(base, rpw)])
    pltpu.sync_copy(rows_buf, rows_hbm.at[pl.ds(base, rpw)])


def _sc_argmax(gt, x16):
    rpw = _NSC // 32
    body = functools.partial(_sc_body, rpw=rpw, n_steps=_SIZE // _LANES)
    return pl.kernel(
        body,
        out_type=[
            jax.ShapeDtypeStruct((_N,), jnp.int32),
            jax.ShapeDtypeStruct((_N,), jnp.int32),
            jax.ShapeDtypeStruct((_N, _GATHER_W), jnp.float32),
        ],
        mesh=plsc.VectorSubcoreMesh(core_axis_name="c", subcore_axis_name="s"),
        scratch_types=[
            pltpu.VMEM((_SIZE,), jnp.float32),
            pltpu.VMEM((_SIZE,), jnp.float32),
            pltpu.VMEM((rpw,), jnp.int32),
            pltpu.VMEM((rpw,), jnp.int32),
            pltpu.VMEM((rpw,), jnp.int32),
            pltpu.VMEM((rpw, _GATHER_W), jnp.float32),
            pltpu.SemaphoreType.DMA,
            pltpu.SemaphoreType.DMA,
            pltpu.SemaphoreType.DMA,
        ],
    )(gt, x16)


def _tc_body(x_ref, gt_ref, s_ref, x0_ref, idx_ref, xat_ref, m_ref,
             *, n_col_blocks, row_block, tc_row_blocks):
    j = pl.program_id(0)
    i = pl.program_id(1)
    rows = pl.ds(i * row_block, row_block)

    x_tile = x_ref[...]
    ts = jnp.sum(x_tile, axis=1, keepdims=True)

    @pl.when(j == 0)
    def _():
        s_ref[rows, :] = ts
        x0_ref[rows, :] = x_tile[:, 0:1]

    @pl.when(j != 0)
    def _():
        s_ref[rows, :] = s_ref[rows, :] + ts

    @pl.when(i < tc_row_blocks)
    def _gt():
        gt_tile = gt_ref[...]
        cols = gt_tile.shape[1]
        tm = jnp.max(gt_tile, axis=1, keepdims=True)
        ta = jnp.argmax(gt_tile, axis=1)[:, None] + j * cols
        onehot = jax.lax.broadcasted_iota(jnp.int32, gt_tile.shape, 1) == (
            ta - j * cols)
        txat = jnp.sum(jnp.where(onehot, x_tile, 0.0), axis=1, keepdims=True)

        @pl.when(j == 0)
        def _():
            m_ref[rows, :] = tm
            idx_ref[rows, :] = ta
            xat_ref[rows, :] = txat

        @pl.when(j != 0)
        def _():
            better = tm > m_ref[rows, :]
            m_ref[rows, :] = jnp.where(better, tm, m_ref[rows, :])
            idx_ref[rows, :] = jnp.where(better, ta, idx_ref[rows, :])
            xat_ref[rows, :] = jnp.where(better, txat, xat_ref[rows, :])


def _tc_sums(x, gt):
    row_block = 256
    col_block = 3200
    n_row_blocks = _N // row_block
    n_col_blocks = _SIZE // col_block
    tc_row_blocks = _NTC // row_block

    body = functools.partial(
        _tc_body,
        n_col_blocks=n_col_blocks,
        row_block=row_block,
        tc_row_blocks=tc_row_blocks,
    )
    full = pl.BlockSpec((_N, 1), lambda j, i: (0, 0))
    return pl.pallas_call(
        body,
        grid=(n_col_blocks, n_row_blocks),
        in_specs=[
            pl.BlockSpec((row_block, col_block), lambda j, i: (i, j)),
            pl.BlockSpec((row_block, col_block),
                         lambda j, i: (jnp.minimum(i, tc_row_blocks - 1), j)),
        ],
        out_specs=[full, full, full, full],
        out_shape=[
            jax.ShapeDtypeStruct((_N, 1), jnp.float32),
            jax.ShapeDtypeStruct((_N, 1), jnp.float32),
            jax.ShapeDtypeStruct((_N, 1), jnp.int32),
            jax.ShapeDtypeStruct((_N, 1), jnp.float32),
        ],
        scratch_shapes=[pltpu.VMEM((_N, 1), jnp.float32)],
    )(x, gt)


def _combine_body(s_ref, x0_ref, idxb_ref, xatb_ref, idxs_ref, lanes_ref,
                  rows_ref, out_ref):
    rowid = jax.lax.broadcasted_iota(jnp.int32, (_N, 1), 0)
    use_sc = rowid >= _NTC
    onehot = jax.lax.broadcasted_iota(jnp.int32, (_N, _GATHER_W), 1) == lanes_ref[...]
    xats = jnp.sum(jnp.where(onehot, rows_ref[...], 0.0), axis=1,
                   keepdims=True)
    idx = jnp.where(use_sc, idxs_ref[...], idxb_ref[...])
    xat = jnp.where(use_sc, xats, xatb_ref[...])
    contrib = (_C1 + _CLOGC) - _EPS * (s_ref[...] - x0_ref[...]) + (
        _EPS - _CONF_F) * xat
    contrib = jnp.where(idx == _PADDING_IDX, 0.0, contrib)
    inv_count = np.float32(1.0 / (_N * _SIZE))
    out_ref[...] = jnp.reshape(jnp.sum(contrib) * inv_count, (1, 1))


def _combine(s, x0, idxb, xatb, idxs, lanes, rows):
    return pl.pallas_call(
        _combine_body,
        out_shape=jax.ShapeDtypeStruct((1, 1), jnp.float32),
    )(s, x0, idxb, xatb, idxs, lanes, rows)


@jax.jit
def kernel(x, gt):
    x16 = jnp.reshape(x, (-1, _GATHER_W))
    idx_sc, lane_sc, rows_sc = _sc_argmax(gt, x16)
    s, x0, idx_b, xat_b = _tc_sums(x, gt)
    out = _combine(s, x0, idx_b, xat_b,
                   jnp.reshape(idx_sc, (_N, 1)),
                   jnp.reshape(lane_sc, (_N, 1)), rows_sc)
    return out[0, 0]


# speedup vs baseline: 1.9384x; 1.8679x over previous
import functools

import jax
import jax.numpy as jnp
import numpy as np
from jax import lax
from jax.experimental import pallas as pl
from jax.experimental.pallas import tpu as pltpu
from jax.experimental.pallas import tpu_sc as plsc

_SIZE = 32000
_N = 4096
_PADDING_IDX = 0
_SMOOTHING = 0.1
_CONFIDENCE = 1.0 - _SMOOTHING
_EPS = np.float32(_SMOOTHING / (_SIZE - 2))
_C1 = np.float32((_SIZE - 2) * float(_EPS) * np.log(float(_EPS)))
_CLOGC = np.float32(_CONFIDENCE * np.log(_CONFIDENCE))
_CONF_F = np.float32(_CONFIDENCE)

_NSC = 1024
_NTC = _N - _NSC
_LANES = 16
_GATHER_W = 128


def _lane_take(v, perm):
    dnums = lax.GatherDimensionNumbers(
        offset_dims=(), collapsed_slice_dims=(0,), start_index_map=(0,))
    return lax.gather(v, perm[:, None], dnums, (1,),
                      mode=lax.GatherScatterMode.PROMISE_IN_BOUNDS)


def _sc_body(gt_hbm, x_hbm, idx_hbm, rows_hbm,
             cbuf0, cbuf1, m_buf, vidx_buf, idx_buf, rows_buf,
             sem0, sem1, gsem, *, rpw, chunk_cols):
    wid = lax.axis_index("s") * 2 + lax.axis_index("c")
    base = _NTC + wid * rpw
    iota = lax.iota(jnp.int32, _LANES)
    neg_inf = jnp.full((_LANES,), -jnp.inf, dtype=jnp.float32)
    zeros_i = jnp.zeros((_LANES,), dtype=jnp.int32)

    n_bands = rpw // 8
    n_chunks = _SIZE // chunk_cols
    n_steps = chunk_cols // _LANES
    bufs = (cbuf0, cbuf1)
    sems = (sem0, sem1)

    sched = [(b, c) for b in range(n_bands) for c in range(n_chunks)]

    def start(k):
        b, c = sched[k]
        return pltpu.async_copy(
            gt_hbm.at[pl.ds(base + b * 8, 8),
                      pl.ds(c * chunk_cols, chunk_cols)],
            bufs[k % 2], sems[k % 2])

    copies = {0: start(0)}
    k = 0
    n_grp = rpw // _LANES
    for half in range(n_grp):
        acc = zeros_i
        for bb in range(2):
            for r8 in range(8):
                m_buf[r8, :] = neg_inf
                vidx_buf[r8, :] = zeros_i
            for c in range(n_chunks):
                if k + 1 < len(sched):
                    copies[k + 1] = start(k + 1)
                copies[k].wait()
                buf = bufs[k % 2]
                iotac = iota + c * chunk_cols

                def row_fn(r8, carry, buf=buf, iotac=iotac):
                    m = m_buf[r8, :]
                    vidx = vidx_buf[r8, :]

                    def step(j, mc):
                        m, vidx = mc
                        v = buf[r8, pl.ds(j * _LANES, _LANES)]
                        cur = iotac + j * _LANES
                        better = v > m
                        return (jnp.where(better, v, m),
                                jnp.where(better, cur, vidx))

                    m, vidx = lax.fori_loop(0, n_steps, step, (m, vidx),
                                            unroll=8)
                    m_buf[r8, :] = m
                    vidx_buf[r8, :] = vidx
                    return carry

                lax.fori_loop(0, 8, row_fn, 0)
                k += 1
            for r8 in range(8):
                m = m_buf[r8, :]
                vidx = vidx_buf[r8, :]
                for kk in (1, 2, 4, 8):
                    perm = jnp.bitwise_xor(iota, kk)
                    m2 = _lane_take(m, perm)
                    v2 = _lane_take(vidx, perm)
                    take = (m2 > m) | ((m2 == m) & (v2 < vidx))
                    m = jnp.where(take, m2, m)
                    vidx = jnp.where(take, v2, vidx)
                acc = jnp.where(iota == bb * 8 + r8, vidx, acc)
        idx_buf[pl.ds(half * _LANES, _LANES)] = acc

    handles = []
    for half in range(n_grp):
        tv = idx_buf[pl.ds(half * _LANES, _LANES)]
        cb = lax.shift_left(lax.shift_right_logical(tv, 7), 7)
        for rr in range(_LANES):
            r = half * _LANES + rr
            row8 = base + (r & ~7)
            handles.append(pltpu.async_copy(
                x_hbm.at[pl.ds(row8, 8),
                         pl.ds(pl.multiple_of(cb[rr], _GATHER_W), _GATHER_W)],
                rows_buf.at[r], gsem))
    for h in handles:
        h.wait()

    pltpu.sync_copy(idx_buf, idx_hbm.at[pl.ds(base, rpw)])
    pltpu.sync_copy(rows_buf, rows_hbm.at[pl.ds(base, rpw)])


def _sc_argmax(gt, x):
    rpw = _NSC // 32
    chunk_cols = 3200
    body = functools.partial(_sc_body, rpw=rpw, chunk_cols=chunk_cols)
    return pl.kernel(
        body,
        out_type=[
            jax.ShapeDtypeStruct((_N,), jnp.int32),
            jax.ShapeDtypeStruct((_N, 8, _GATHER_W), jnp.float32),
        ],
        mesh=plsc.VectorSubcoreMesh(core_axis_name="c", subcore_axis_name="s"),
        scratch_types=[
            pltpu.VMEM((8, chunk_cols), jnp.float32),
            pltpu.VMEM((8, chunk_cols), jnp.float32),
            pltpu.VMEM((8, _LANES), jnp.float32),
            pltpu.VMEM((8, _LANES), jnp.int32),
            pltpu.VMEM((rpw,), jnp.int32),
            pltpu.VMEM((rpw, 8, _GATHER_W), jnp.float32),
            pltpu.SemaphoreType.DMA,
            pltpu.SemaphoreType.DMA,
            pltpu.SemaphoreType.DMA,
        ],
    )(gt, x)


def _tc_body(x_ref, gt_ref, s_ref, x0_ref, idx_ref, xat_ref, m_ref,
             *, n_col_blocks, row_block, tc_row_blocks):
    j = pl.program_id(0)
    i = pl.program_id(1)
    rows = pl.ds(i * row_block, row_block)

    x_tile = x_ref[...]
    ts = jnp.sum(x_tile, axis=1, keepdims=True)

    @pl.when(j == 0)
    def _():
        s_ref[rows, :] = ts
        x0_ref[rows, :] = x_tile[:, 0:1]

    @pl.when(j != 0)
    def _():
        s_ref[rows, :] = s_ref[rows, :] + ts

    @pl.when(i < tc_row_blocks)
    def _gt():
        gt_tile = gt_ref[...]
        cols = gt_tile.shape[1]
        tm = jnp.max(gt_tile, axis=1, keepdims=True)
        ta = jnp.argmax(gt_tile, axis=1)[:, None] + j * cols
        onehot = jax.lax.broadcasted_iota(jnp.int32, gt_tile.shape, 1) == (
            ta - j * cols)
        txat = jnp.sum(jnp.where(onehot, x_tile, 0.0), axis=1, keepdims=True)

        @pl.when(j == 0)
        def _():
            m_ref[rows, :] = tm
            idx_ref[rows, :] = ta
            xat_ref[rows, :] = txat

        @pl.when(j != 0)
        def _():
            better = tm > m_ref[rows, :]
            m_ref[rows, :] = jnp.where(better, tm, m_ref[rows, :])
            idx_ref[rows, :] = jnp.where(better, ta, idx_ref[rows, :])
            xat_ref[rows, :] = jnp.where(better, txat, xat_ref[rows, :])


def _tc_sums(x, gt):
    row_block = 256
    col_block = 3200
    n_row_blocks = _N // row_block
    n_col_blocks = _SIZE // col_block
    tc_row_blocks = _NTC // row_block

    body = functools.partial(
        _tc_body,
        n_col_blocks=n_col_blocks,
        row_block=row_block,
        tc_row_blocks=tc_row_blocks,
    )
    full = pl.BlockSpec((_N, 1), lambda j, i: (0, 0))
    return pl.pallas_call(
        body,
        grid=(n_col_blocks, n_row_blocks),
        in_specs=[
            pl.BlockSpec((row_block, col_block), lambda j, i: (i, j)),
            pl.BlockSpec((row_block, col_block),
                         lambda j, i: (jnp.minimum(i, tc_row_blocks - 1), j)),
        ],
        out_specs=[full, full, full, full],
        out_shape=[
            jax.ShapeDtypeStruct((_N, 1), jnp.float32),
            jax.ShapeDtypeStruct((_N, 1), jnp.float32),
            jax.ShapeDtypeStruct((_N, 1), jnp.int32),
            jax.ShapeDtypeStruct((_N, 1), jnp.float32),
        ],
        scratch_shapes=[pltpu.VMEM((_N, 1), jnp.float32)],
    )(x, gt)


def _combine_body(s_ref, x0_ref, idxb_ref, xatb_ref, idxs_ref,
                  rows_ref, out_ref):
    rowid = jax.lax.broadcasted_iota(jnp.int32, (_N, 1), 0)
    use_sc = rowid >= _NTC
    lanes = jnp.bitwise_and(idxs_ref[...], _GATHER_W - 1)[:, None, :]
    subs = jnp.bitwise_and(rowid, 7)[:, None, :]
    sh = (_N, 8, _GATHER_W)
    onehot = (jax.lax.broadcasted_iota(jnp.int32, sh, 2) == lanes) & (
        jax.lax.broadcasted_iota(jnp.int32, sh, 1) == subs)
    xats = jnp.sum(jnp.where(onehot, rows_ref[...], 0.0),
                   axis=(1, 2))[:, None]
    idx = jnp.where(use_sc, idxs_ref[...], idxb_ref[...])
    xat = jnp.where(use_sc, xats, xatb_ref[...])
    contrib = (_C1 + _CLOGC) - _EPS * (s_ref[...] - x0_ref[...]) + (
        _EPS - _CONF_F) * xat
    contrib = jnp.where(idx == _PADDING_IDX, 0.0, contrib)
    inv_count = np.float32(1.0 / (_N * _SIZE))
    out_ref[...] = jnp.reshape(jnp.sum(contrib) * inv_count, (1, 1))


def _combine(s, x0, idxb, xatb, idxs, rows):
    return pl.pallas_call(
        _combine_body,
        out_shape=jax.ShapeDtypeStruct((1, 1), jnp.float32),
    )(s, x0, idxb, xatb, idxs, rows)


@jax.jit
def kernel(x, gt):
    idx_sc, rows_sc = _sc_argmax(gt, x)
    s, x0, idx_b, xat_b = _tc_sums(x, gt)
    out = _combine(s, x0, idx_b, xat_b,
                   jnp.reshape(idx_sc, (_N, 1)), rows_sc)
    return out[0, 0]


# speedup vs baseline: 2.1127x; 1.0899x over previous
import functools

import jax
import jax.numpy as jnp
import numpy as np
from jax import lax
from jax.experimental import pallas as pl
from jax.experimental.pallas import tpu as pltpu
from jax.experimental.pallas import tpu_sc as plsc

_SIZE = 32000
_N = 4096
_PADDING_IDX = 0
_SMOOTHING = 0.1
_CONFIDENCE = 1.0 - _SMOOTHING
_EPS = np.float32(_SMOOTHING / (_SIZE - 2))
_C1 = np.float32((_SIZE - 2) * float(_EPS) * np.log(float(_EPS)))
_CLOGC = np.float32(_CONFIDENCE * np.log(_CONFIDENCE))
_CONF_F = np.float32(_CONFIDENCE)

_NSC = 1024
_NTC = _N - _NSC
_LANES = 16
_GATHER_W = 128


def _lane_take(v, perm):
    dnums = lax.GatherDimensionNumbers(
        offset_dims=(), collapsed_slice_dims=(0,), start_index_map=(0,))
    return lax.gather(v, perm[:, None], dnums, (1,),
                      mode=lax.GatherScatterMode.PROMISE_IN_BOUNDS)


def _sc_body(gt_hbm, x_hbm, idx_hbm, rows_hbm,
             cbuf0, cbuf1, m_buf, vidx_buf, idx_buf, rows_buf,
             sem0, sem1, gsem, *, rpw, chunk_cols):
    wid = lax.axis_index("s") * 2 + lax.axis_index("c")
    base = _NTC + wid * rpw
    iota = lax.iota(jnp.int32, _LANES)
    neg_inf = jnp.full((_LANES,), -jnp.inf, dtype=jnp.float32)
    zeros_i = jnp.zeros((_LANES,), dtype=jnp.int32)

    n_bands = rpw // 8
    n_chunks = _SIZE // chunk_cols
    n_steps = chunk_cols // _LANES
    bufs = (cbuf0, cbuf1)
    sems = (sem0, sem1)

    sched = [(b, c) for b in range(n_bands) for c in range(n_chunks)]

    def start(k):
        b, c = sched[k]
        return pltpu.async_copy(
            gt_hbm.at[pl.ds(base + b * 8, 8),
                      pl.ds(c * chunk_cols, chunk_cols)],
            bufs[k % 2], sems[k % 2])

    copies = {0: start(0)}
    k = 0
    n_grp = rpw // _LANES
    for half in range(n_grp):
        acc = zeros_i
        for bb in range(2):
            for r8 in range(8):
                m_buf[r8, :] = neg_inf
                vidx_buf[r8, :] = zeros_i
            for c in range(n_chunks):
                if k + 1 < len(sched):
                    copies[k + 1] = start(k + 1)
                copies[k].wait()
                buf = bufs[k % 2]
                iotac = iota + c * chunk_cols

                def row_fn(r8, carry, buf=buf, iotac=iotac):
                    m = m_buf[r8, :]
                    vidx = vidx_buf[r8, :]

                    def step(j, mc):
                        m, vidx = mc
                        v = buf[r8, pl.ds(j * _LANES, _LANES)]
                        cur = iotac + j * _LANES
                        better = v > m
                        return (jnp.where(better, v, m),
                                jnp.where(better, cur, vidx))

                    m, vidx = lax.fori_loop(0, n_steps, step, (m, vidx),
                                            unroll=8)
                    m_buf[r8, :] = m
                    vidx_buf[r8, :] = vidx
                    return carry

                lax.fori_loop(0, 8, row_fn, 0)
                k += 1
            for r8 in range(8):
                m = m_buf[r8, :]
                vidx = vidx_buf[r8, :]
                for kk in (1, 2, 4, 8):
                    perm = jnp.bitwise_xor(iota, kk)
                    m2 = _lane_take(m, perm)
                    v2 = _lane_take(vidx, perm)
                    take = (m2 > m) | ((m2 == m) & (v2 < vidx))
                    m = jnp.where(take, m2, m)
                    vidx = jnp.where(take, v2, vidx)
                acc = jnp.where(iota == bb * 8 + r8, vidx, acc)
        idx_buf[pl.ds(half * _LANES, _LANES)] = acc

    handles = []
    for half in range(n_grp):
        tv = idx_buf[pl.ds(half * _LANES, _LANES)]
        cb = lax.shift_left(lax.shift_right_logical(tv, 7), 7)
        for rr in range(_LANES):
            r = half * _LANES + rr
            row8 = base + (r & ~7)
            handles.append(pltpu.async_copy(
                x_hbm.at[pl.ds(row8, 8),
                         pl.ds(pl.multiple_of(cb[rr], _GATHER_W), _GATHER_W)],
                rows_buf.at[r], gsem))
    for h in handles:
        h.wait()

    pltpu.sync_copy(idx_buf, idx_hbm.at[pl.ds(base, rpw)])
    pltpu.sync_copy(rows_buf, rows_hbm.at[pl.ds(base, rpw)])


def _sc_argmax(gt, x):
    rpw = _NSC // 32
    chunk_cols = 3200
    body = functools.partial(_sc_body, rpw=rpw, chunk_cols=chunk_cols)
    return pl.kernel(
        body,
        out_type=[
            jax.ShapeDtypeStruct((_N,), jnp.int32),
            jax.ShapeDtypeStruct((_N, 8, _GATHER_W), jnp.float32),
        ],
        mesh=plsc.VectorSubcoreMesh(core_axis_name="c", subcore_axis_name="s"),
        scratch_types=[
            pltpu.VMEM((8, chunk_cols), jnp.float32),
            pltpu.VMEM((8, chunk_cols), jnp.float32),
            pltpu.VMEM((8, _LANES), jnp.float32),
            pltpu.VMEM((8, _LANES), jnp.int32),
            pltpu.VMEM((rpw,), jnp.int32),
            pltpu.VMEM((rpw, 8, _GATHER_W), jnp.float32),
            pltpu.SemaphoreType.DMA,
            pltpu.SemaphoreType.DMA,
            pltpu.SemaphoreType.DMA,
        ],
    )(gt, x)


def _tc_body(x_ref, gt_ref, s_ref, x0_ref, idx_ref, xat_ref, m_ref,
             *, n_col_blocks, row_block, tc_row_blocks):
    j = pl.program_id(0)
    i = pl.program_id(1)
    rows = pl.ds(i * row_block, row_block)

    x_tile = x_ref[...]
    ts = jnp.sum(x_tile, axis=1, keepdims=True)

    @pl.when(j == 0)
    def _():
        s_ref[rows, :] = ts
        x0_ref[rows, :] = x_tile[:, 0:1]

    @pl.when(j != 0)
    def _():
        s_ref[rows, :] = s_ref[rows, :] + ts

    @pl.when(i < tc_row_blocks)
    def _gt():
        gt_tile = gt_ref[...]
        cols = gt_tile.shape[1]
        tm = jnp.max(gt_tile, axis=1, keepdims=True)
        ta = jnp.argmax(gt_tile, axis=1)[:, None] + j * cols
        onehot = jax.lax.broadcasted_iota(jnp.int32, gt_tile.shape, 1) == (
            ta - j * cols)
        txat = jnp.sum(jnp.where(onehot, x_tile, 0.0), axis=1, keepdims=True)

        @pl.when(j == 0)
        def _():
            m_ref[rows, :] = tm
            idx_ref[rows, :] = ta
            xat_ref[rows, :] = txat

        @pl.when(j != 0)
        def _():
            better = tm > m_ref[rows, :]
            m_ref[rows, :] = jnp.where(better, tm, m_ref[rows, :])
            idx_ref[rows, :] = jnp.where(better, ta, idx_ref[rows, :])
            xat_ref[rows, :] = jnp.where(better, txat, xat_ref[rows, :])


def _tc_sums(x, gt):
    row_block = 512
    col_block = 3200
    n_row_blocks = _N // row_block
    n_col_blocks = _SIZE // col_block
    tc_row_blocks = _NTC // row_block

    body = functools.partial(
        _tc_body,
        n_col_blocks=n_col_blocks,
        row_block=row_block,
        tc_row_blocks=tc_row_blocks,
    )
    full = pl.BlockSpec((_N, 1), lambda j, i: (0, 0))
    return pl.pallas_call(
        body,
        grid=(n_col_blocks, n_row_blocks),
        in_specs=[
            pl.BlockSpec((row_block, col_block), lambda j, i: (i, j)),
            pl.BlockSpec((row_block, col_block),
                         lambda j, i: (jnp.minimum(i, tc_row_blocks - 1), j)),
        ],
        out_specs=[full, full, full, full],
        out_shape=[
            jax.ShapeDtypeStruct((_N, 1), jnp.float32),
            jax.ShapeDtypeStruct((_N, 1), jnp.float32),
            jax.ShapeDtypeStruct((_N, 1), jnp.int32),
            jax.ShapeDtypeStruct((_N, 1), jnp.float32),
        ],
        scratch_shapes=[pltpu.VMEM((_N, 1), jnp.float32)],
    )(x, gt)


def _combine_body(s_ref, x0_ref, idxb_ref, xatb_ref, idxs_ref,
                  rows_ref, out_ref):
    rowid = jax.lax.broadcasted_iota(jnp.int32, (_N, 1), 0)
    use_sc = rowid >= _NTC
    lanes = jnp.bitwise_and(idxs_ref[...], _GATHER_W - 1)[:, None, :]
    subs = jnp.bitwise_and(rowid, 7)[:, None, :]
    sh = (_N, 8, _GATHER_W)
    onehot = (jax.lax.broadcasted_iota(jnp.int32, sh, 2) == lanes) & (
        jax.lax.broadcasted_iota(jnp.int32, sh, 1) == subs)
    xats = jnp.sum(jnp.where(onehot, rows_ref[...], 0.0),
                   axis=(1, 2))[:, None]
    idx = jnp.where(use_sc, idxs_ref[...], idxb_ref[...])
    xat = jnp.where(use_sc, xats, xatb_ref[...])
    contrib = (_C1 + _CLOGC) - _EPS * (s_ref[...] - x0_ref[...]) + (
        _EPS - _CONF_F) * xat
    contrib = jnp.where(idx == _PADDING_IDX, 0.0, contrib)
    inv_count = np.float32(1.0 / (_N * _SIZE))
    out_ref[...] = jnp.reshape(jnp.sum(contrib) * inv_count, (1, 1))


def _combine(s, x0, idxb, xatb, idxs, rows):
    return pl.pallas_call(
        _combine_body,
        out_shape=jax.ShapeDtypeStruct((1, 1), jnp.float32),
    )(s, x0, idxb, xatb, idxs, rows)


@jax.jit
def kernel(x, gt):
    idx_sc, rows_sc = _sc_argmax(gt, x)
    s, x0, idx_b, xat_b = _tc_sums(x, gt)
    out = _combine(s, x0, idx_b, xat_b,
                   jnp.reshape(idx_sc, (_N, 1)), rows_sc)
    return out[0, 0]


# speedup vs baseline: 2.1662x; 1.0253x over previous
import functools

import jax
import jax.numpy as jnp
import numpy as np
from jax import lax
from jax.experimental import pallas as pl
from jax.experimental.pallas import tpu as pltpu
from jax.experimental.pallas import tpu_sc as plsc

_SIZE = 32000
_N = 4096
_PADDING_IDX = 0
_SMOOTHING = 0.1
_CONFIDENCE = 1.0 - _SMOOTHING
_EPS = np.float32(_SMOOTHING / (_SIZE - 2))
_C1 = np.float32((_SIZE - 2) * float(_EPS) * np.log(float(_EPS)))
_CLOGC = np.float32(_CONFIDENCE * np.log(_CONFIDENCE))
_CONF_F = np.float32(_CONFIDENCE)

_NSC = 2560
_NTC = _N - _NSC
_LANES = 16
_GATHER_W = 128


def _lane_take(v, perm):
    dnums = lax.GatherDimensionNumbers(
        offset_dims=(), collapsed_slice_dims=(0,), start_index_map=(0,))
    return lax.gather(v, perm[:, None], dnums, (1,),
                      mode=lax.GatherScatterMode.PROMISE_IN_BOUNDS)


def _sc_body(gt_hbm, x_hbm, idx_hbm, rows_hbm,
             cbuf0, cbuf1, m_buf, vidx_buf, idx_buf, gblk_buf, rows_buf,
             sem0, sem1, gsem, *, rpw, chunk_cols):
    wid = lax.axis_index("s") * 2 + lax.axis_index("c")
    base = _NTC + wid * rpw
    iota = lax.iota(jnp.int32, _LANES)
    neg_inf = jnp.full((_LANES,), -jnp.inf, dtype=jnp.float32)
    zeros_i = jnp.zeros((_LANES,), dtype=jnp.int32)

    n_bands = rpw // 8
    n_chunks = _SIZE // chunk_cols
    n_steps = chunk_cols // _LANES
    bufs = (cbuf0, cbuf1)
    sems = (sem0, sem1)

    sched = [(b, c) for b in range(n_bands) for c in range(n_chunks)]

    def start(k):
        b, c = sched[k]
        return pltpu.async_copy(
            gt_hbm.at[pl.ds(base + b * 8, 8),
                      pl.ds(c * chunk_cols, chunk_cols)],
            bufs[k % 2], sems[k % 2])

    copies = {0: start(0)}
    k = 0
    n_grp = rpw // _LANES
    for half in range(n_grp):
        acc = zeros_i
        for bb in range(2):
            for r8 in range(8):
                m_buf[r8, :] = neg_inf
                vidx_buf[r8, :] = zeros_i
            for c in range(n_chunks):
                if k + 1 < len(sched):
                    copies[k + 1] = start(k + 1)
                copies[k].wait()
                buf = bufs[k % 2]
                iotac = iota + c * chunk_cols

                def row_fn(r8, carry, buf=buf, iotac=iotac):
                    m = m_buf[r8, :]
                    vidx = vidx_buf[r8, :]

                    def step(j, mc):
                        m, vidx = mc
                        v = buf[r8, pl.ds(j * _LANES, _LANES)]
                        cur = iotac + j * _LANES
                        better = v > m
                        return (jnp.where(better, v, m),
                                jnp.where(better, cur, vidx))

                    m, vidx = lax.fori_loop(0, n_steps, step, (m, vidx),
                                            unroll=4)
                    m_buf[r8, :] = m
                    vidx_buf[r8, :] = vidx
                    return carry

                lax.fori_loop(0, 8, row_fn, 0)
                k += 1
            for r8 in range(8):
                m = m_buf[r8, :]
                vidx = vidx_buf[r8, :]
                for kk in (1, 2, 4, 8):
                    perm = jnp.bitwise_xor(iota, kk)
                    m2 = _lane_take(m, perm)
                    v2 = _lane_take(vidx, perm)
                    take = (m2 > m) | ((m2 == m) & (v2 < vidx))
                    m = jnp.where(take, m2, m)
                    vidx = jnp.where(take, v2, vidx)
                acc = jnp.where(iota == bb * 8 + r8, vidx, acc)
        idx_buf[pl.ds(half * _LANES, _LANES)] = acc

    ring = 8
    pend = []

    def drain_one():
        h, r0, sub0, w0, slot0 = pend.pop(0)
        h.wait()
        v16 = gblk_buf[slot0, sub0,
                       pl.ds(pl.multiple_of(w0 * _LANES, _LANES), _LANES)]
        rows_buf[r0, :] = v16

    for half in range(n_grp):
        tv = idx_buf[pl.ds(half * _LANES, _LANES)]
        cb = lax.shift_left(lax.shift_right_logical(tv, 7), 7)
        wv = lax.shift_right_logical(
            lax.bitwise_and(tv, _GATHER_W - 1), 4)
        for rr in range(_LANES):
            r = half * _LANES + rr
            slot = r % ring
            if len(pend) >= ring:
                drain_one()
            h = pltpu.async_copy(
                x_hbm.at[pl.ds(base + (r & ~7), 8),
                         pl.ds(pl.multiple_of(cb[rr], _GATHER_W), _GATHER_W)],
                gblk_buf.at[slot], gsem)
            pend.append((h, r, r & 7, wv[rr], slot))
    while pend:
        drain_one()

    pltpu.sync_copy(idx_buf, idx_hbm.at[pl.ds(base, rpw)])
    pltpu.sync_copy(rows_buf, rows_hbm.at[pl.ds(base, rpw)])


def _sc_argmax(gt, x):
    rpw = _NSC // 32
    chunk_cols = 6400
    body = functools.partial(_sc_body, rpw=rpw, chunk_cols=chunk_cols)
    return pl.kernel(
        body,
        out_type=[
            jax.ShapeDtypeStruct((_N,), jnp.int32),
            jax.ShapeDtypeStruct((_N, _LANES), jnp.float32),
        ],
        mesh=plsc.VectorSubcoreMesh(core_axis_name="c", subcore_axis_name="s"),
        scratch_types=[
            pltpu.VMEM((8, chunk_cols), jnp.float32),
            pltpu.VMEM((8, chunk_cols), jnp.float32),
            pltpu.VMEM((8, _LANES), jnp.float32),
            pltpu.VMEM((8, _LANES), jnp.int32),
            pltpu.VMEM((rpw,), jnp.int32),
            pltpu.VMEM((8, 8, _GATHER_W), jnp.float32),
            pltpu.VMEM((rpw, _LANES), jnp.float32),
            pltpu.SemaphoreType.DMA,
            pltpu.SemaphoreType.DMA,
            pltpu.SemaphoreType.DMA,
        ],
    )(gt, x)


def _tc_body(x_ref, gt_ref, s_ref, x0_ref, idx_ref, xat_ref, m_ref,
             *, n_col_blocks, row_block, tc_row_blocks):
    j = pl.program_id(0)
    i = pl.program_id(1)
    rows = pl.ds(i * row_block, row_block)

    x_tile = x_ref[...]
    ts = jnp.sum(x_tile, axis=1, keepdims=True)

    @pl.when(j == 0)
    def _():
        s_ref[rows, :] = ts
        x0_ref[rows, :] = x_tile[:, 0:1]

    @pl.when(j != 0)
    def _():
        s_ref[rows, :] = s_ref[rows, :] + ts

    @pl.when(i < tc_row_blocks)
    def _gt():
        gt_tile = gt_ref[...]
        cols = gt_tile.shape[1]
        tm = jnp.max(gt_tile, axis=1, keepdims=True)
        ta = jnp.argmax(gt_tile, axis=1)[:, None] + j * cols
        onehot = jax.lax.broadcasted_iota(jnp.int32, gt_tile.shape, 1) == (
            ta - j * cols)
        txat = jnp.sum(jnp.where(onehot, x_tile, 0.0), axis=1, keepdims=True)

        @pl.when(j == 0)
        def _():
            m_ref[rows, :] = tm
            idx_ref[rows, :] = ta
            xat_ref[rows, :] = txat

        @pl.when(j != 0)
        def _():
            better = tm > m_ref[rows, :]
            m_ref[rows, :] = jnp.where(better, tm, m_ref[rows, :])
            idx_ref[rows, :] = jnp.where(better, ta, idx_ref[rows, :])
            xat_ref[rows, :] = jnp.where(better, txat, xat_ref[rows, :])


def _tc_sums(x, gt):
    row_block = 512
    col_block = 3200
    n_row_blocks = _N // row_block
    n_col_blocks = _SIZE // col_block
    tc_row_blocks = _NTC // row_block

    body = functools.partial(
        _tc_body,
        n_col_blocks=n_col_blocks,
        row_block=row_block,
        tc_row_blocks=tc_row_blocks,
    )
    full = pl.BlockSpec((_N, 1), lambda j, i: (0, 0))
    return pl.pallas_call(
        body,
        grid=(n_col_blocks, n_row_blocks),
        in_specs=[
            pl.BlockSpec((row_block, col_block), lambda j, i: (i, j)),
            pl.BlockSpec((row_block, col_block),
                         lambda j, i: (jnp.minimum(i, tc_row_blocks - 1), j)),
        ],
        out_specs=[full, full, full, full],
        out_shape=[
            jax.ShapeDtypeStruct((_N, 1), jnp.float32),
            jax.ShapeDtypeStruct((_N, 1), jnp.float32),
            jax.ShapeDtypeStruct((_N, 1), jnp.int32),
            jax.ShapeDtypeStruct((_N, 1), jnp.float32),
        ],
        scratch_shapes=[pltpu.VMEM((_N, 1), jnp.float32)],
    )(x, gt)


def _combine_body(s_ref, x0_ref, idxb_ref, xatb_ref, idxs_ref,
                  rows_ref, out_ref):
    rowid = jax.lax.broadcasted_iota(jnp.int32, (_N, 1), 0)
    use_sc = rowid >= _NTC
    lanes = jnp.bitwise_and(idxs_ref[...], _LANES - 1)
    onehot = jax.lax.broadcasted_iota(jnp.int32, (_N, _LANES), 1) == lanes
    xats = jnp.sum(jnp.where(onehot, rows_ref[...], 0.0), axis=1,
                   keepdims=True)
    idx = jnp.where(use_sc, idxs_ref[...], idxb_ref[...])
    xat = jnp.where(use_sc, xats, xatb_ref[...])
    contrib = (_C1 + _CLOGC) - _EPS * (s_ref[...] - x0_ref[...]) + (
        _EPS - _CONF_F) * xat
    contrib = jnp.where(idx == _PADDING_IDX, 0.0, contrib)
    inv_count = np.float32(1.0 / (_N * _SIZE))
    out_ref[...] = jnp.reshape(jnp.sum(contrib) * inv_count, (1, 1))


def _combine(s, x0, idxb, xatb, idxs, rows):
    return pl.pallas_call(
        _combine_body,
        out_shape=jax.ShapeDtypeStruct((1, 1), jnp.float32),
    )(s, x0, idxb, xatb, idxs, rows)


@jax.jit
def kernel(x, gt):
    idx_sc, rows_sc = _sc_argmax(gt, x)
    s, x0, idx_b, xat_b = _tc_sums(x, gt)
    out = _combine(s, x0, idx_b, xat_b,
                   jnp.reshape(idx_sc, (_N, 1)), rows_sc)
    return out[0, 0]
